# baseline (speedup 1.0000x reference)
import functools

import jax
import jax.numpy as jnp
from jax import lax
from jax.experimental import pallas as pl
from jax.experimental.pallas import tpu as pltpu
from jax.experimental.pallas import tpu_sc as plsc

_BM = 400

_SC_ROWS = 2560
_NW = 32
_RPW = _SC_ROWS // _NW
_CH = 8


def _gconv_kernel(adj_ref, x_ref, w_ref, b_ref, out_ref):
    t = jnp.dot(
        adj_ref[...].astype(jnp.bfloat16),
        x_ref[...].astype(jnp.bfloat16),
        preferred_element_type=jnp.float32,
    )
    out_ref[...] = (
        jnp.dot(t, w_ref[...], preferred_element_type=jnp.float32) + b_ref[...]
    )


def _sc_probe(adj):
    mesh = plsc.VectorSubcoreMesh(core_axis_name="c", subcore_axis_name="s")

    @functools.partial(
        pl.kernel,
        mesh=mesh,
        out_type=jax.ShapeDtypeStruct((_NW, 16), jnp.float32),
        scratch_types=[pltpu.VMEM((_CH, 10000), jnp.float32)],
    )
    def k(adj_hbm, out_hbm, buf):
        wid = lax.axis_index("s") * 2 + lax.axis_index("c")
        base = wid * _RPW
        for c in range(_RPW // _CH):
            pltpu.sync_copy(adj_hbm.at[pl.ds(base + c * _CH, _CH)], buf)
        pltpu.sync_copy(buf.at[0, pl.ds(0, 16)], out_hbm.at[wid])

    return k(adj)


@jax.jit
def kernel(input, adj, weight, bias):
    n, d_in = input.shape
    d_out = weight.shape[1]
    m = adj.shape[0]
    probe = _sc_probe(adj)
    out = pl.pallas_call(
        _gconv_kernel,
        grid=(m // _BM,),
        in_specs=[
            pl.BlockSpec((_BM, n), lambda i: (i, 0)),
            pl.BlockSpec((n, d_in), lambda i: (0, 0)),
            pl.BlockSpec((d_in, d_out), lambda i: (0, 0)),
            pl.BlockSpec((1, d_out), lambda i: (0, 0)),
        ],
        out_specs=pl.BlockSpec((_BM, d_out), lambda i: (i, 0)),
        out_shape=jax.ShapeDtypeStruct((m, d_out), jnp.float32),
        compiler_params=pltpu.CompilerParams(
            dimension_semantics=("arbitrary",),
            vmem_limit_bytes=120 * 1024 * 1024,
        ),
    )(adj, input, weight, bias)
    return out + 0.0 * probe[0, 0]


# speedup vs baseline: 1.4457x; 1.4457x over previous
import jax
import jax.numpy as jnp
from jax.experimental import pallas as pl
from jax.experimental.pallas import tpu as pltpu

_BM = 400


def _gconv_kernel(adj_ref, x_ref, w_ref, b_ref, out_ref):
    t = jnp.dot(
        adj_ref[...].astype(jnp.bfloat16),
        x_ref[...].astype(jnp.bfloat16),
        preferred_element_type=jnp.float32,
    )
    out_ref[...] = (
        jnp.dot(t, w_ref[...], preferred_element_type=jnp.float32) + b_ref[...]
    )


@jax.jit
def kernel(input, adj, weight, bias):
    n, d_in = input.shape
    d_out = weight.shape[1]
    m = adj.shape[0]
    return pl.pallas_call(
        _gconv_kernel,
        grid=(m // _BM,),
        in_specs=[
            pl.BlockSpec((_BM, n), lambda i: (i, 0)),
            pl.BlockSpec((n, d_in), lambda i: (0, 0)),
            pl.BlockSpec((d_in, d_out), lambda i: (0, 0)),
            pl.BlockSpec((1, d_out), lambda i: (0, 0)),
        ],
        out_specs=pl.BlockSpec((_BM, d_out), lambda i: (i, 0)),
        out_shape=jax.ShapeDtypeStruct((m, d_out), jnp.float32),
        compiler_params=pltpu.CompilerParams(
            dimension_semantics=("arbitrary",),
            vmem_limit_bytes=120 * 1024 * 1024,
        ),
    )(adj, input, weight, bias)
